# Initial kernel scaffold; baseline (speedup 1.0000x reference)
import jax
import jax.numpy as jnp
from jax.experimental import pallas as pl


def kernel(x, edge_index, edge_type, edge_score, batch, pre_W, pre_b, pre_bn_g, pre_bn_b, conv_W, conv_b, skip_W, skip_b, layer_bn_g, layer_bn_b, post_W1, post_b1, post_bn1_g, post_bn1_b, post_W2, post_b2, post_bn2_g, post_bn2_b, post_W3, post_b3):
    raise NotImplementedError("write your pallas kernel here")


# speedup vs baseline: 9.1405x; 9.1405x over previous
import jax
import jax.numpy as jnp
from jax import lax
from jax.experimental import pallas as pl
from jax.experimental.pallas import tpu as pltpu
from jax.experimental.pallas import tpu_sc as plsc

N = 10000
E = 320000
NREL = 3
D = 128
NG = 64
EPS = 1e-5

NC = 2
NS = 16
LANES = 16
HALF = N // NC
DUMMY = NREL * HALF
SPAD = 15104
RPT = SPAD // NS
CHUNK = 64
ET = 353280
PADE = ET - E - NREL * N
ETR = ET // 128
EPT = ET // NS
EPW = ET // (NC * NS)

_SC_PARAMS = dict(
    compiler_params=pltpu.CompilerParams(needs_layout_passes=False),
)


def _sc_mesh():
    return plsc.VectorSubcoreMesh(core_axis_name="c", subcore_axis_name="s")


def _deg_body(dst_hbm, et_hbm, es_hbm, part_hbm, dstb, etb, esb, accb):
    c = lax.axis_index("c")
    s = lax.axis_index("s")
    wid = c * NS + s
    base = wid * EPW
    pltpu.sync_copy(dst_hbm.at[pl.ds(base, EPW)], dstb)
    pltpu.sync_copy(et_hbm.at[pl.ds(base, EPW)], etb)
    pltpu.sync_copy(es_hbm.at[pl.ds(base, EPW)], esb)
    zv = jnp.zeros((LANES,), jnp.float32)

    def _zero(i, carry):
        accb[pl.ds(i * LANES, LANES)] = zv
        return carry

    lax.fori_loop(0, NREL * N // LANES, _zero, 0)

    def _edge(i, carry):
        t = etb[pl.ds(i * LANES, LANES)]
        d = dstb[pl.ds(i * LANES, LANES)]
        w = esb[pl.ds(i * LANES, LANES)]
        plsc.addupdate_scatter(accb, [t * N + d], w)
        return carry

    lax.fori_loop(0, EPW // LANES, _edge, 0)
    pltpu.sync_copy(accb, part_hbm.at[wid])


def _deg_call(dst_x, et_x, es_x):
    return pl.kernel(
        _deg_body,
        out_type=jax.ShapeDtypeStruct((NC * NS, NREL * N), jnp.float32),
        mesh=_sc_mesh(),
        scratch_types=[
            pltpu.VMEM((EPW,), jnp.int32),
            pltpu.VMEM((EPW,), jnp.int32),
            pltpu.VMEM((EPW,), jnp.float32),
            pltpu.VMEM((NREL * N,), jnp.float32),
        ],
        **_SC_PARAMS,
    )(dst_x, et_x, es_x)


def _w_body(dinv_hbm, gs_hbm, gd_hbm, es_hbm, w_hbm, dinvb, gsb, gdb, esb, wb):
    c = lax.axis_index("c")
    s = lax.axis_index("s")
    base = (c * NS + s) * EPW
    pltpu.sync_copy(dinv_hbm, dinvb)
    pltpu.sync_copy(gs_hbm.at[pl.ds(base, EPW)], gsb)
    pltpu.sync_copy(gd_hbm.at[pl.ds(base, EPW)], gdb)
    pltpu.sync_copy(es_hbm.at[pl.ds(base, EPW)], esb)

    def _edge(i, carry):
        o = i * LANES
        a = plsc.load_gather(dinvb, [gsb[pl.ds(o, LANES)]])
        b = plsc.load_gather(dinvb, [gdb[pl.ds(o, LANES)]])
        wb[pl.ds(o, LANES)] = esb[pl.ds(o, LANES)] * a * b
        return carry

    lax.fori_loop(0, EPW // LANES, _edge, 0)
    pltpu.sync_copy(wb, w_hbm.at[pl.ds(base, EPW)])


def _w_call(dinv_flat, gs_x, gd_x, es_x):
    return pl.kernel(
        _w_body,
        out_type=jax.ShapeDtypeStruct((ET,), jnp.float32),
        mesh=_sc_mesh(),
        scratch_types=[
            pltpu.VMEM((NREL * N,), jnp.float32),
            pltpu.VMEM((EPW,), jnp.int32),
            pltpu.VMEM((EPW,), jnp.int32),
            pltpu.VMEM((EPW,), jnp.float32),
            pltpu.VMEM((EPW,), jnp.float32),
        ],
        **_SC_PARAMS,
    )(dinv_flat, gs_x, gd_x, es_x)


def _msg_body(tab_hbm, gs_hbm, row_hbm, w_hbm, s_hbm, gb, rb, sb, rows, acc, sem):
    c = lax.axis_index("c")
    s = lax.axis_index("s")
    zv = jnp.zeros((LANES,), jnp.float32)

    def _zrow(j, carry):
        for q in range(D // LANES):
            rows[j, pl.ds(q * LANES, LANES)] = zv
        return carry

    lax.fori_loop(0, CHUNK, _zrow, 0)

    def _zstripe(i, carry):
        pltpu.sync_copy(rows, acc.at[pl.ds(s * RPT + i * CHUNK, CHUNK)])
        return carry

    lax.fori_loop(0, RPT // CHUNK, _zstripe, 0)
    pltpu.sync_copy(rows.at[pl.ds(0, RPT % CHUNK)],
                    acc.at[pl.ds(s * RPT + (RPT // CHUNK) * CHUNK, RPT % CHUNK)])
    plsc.subcore_barrier()

    def _chunk(ci, carry):
        off = s * EPT + ci * CHUNK
        pltpu.sync_copy(gs_hbm.at[pl.ds(off, CHUNK)], gb)
        pltpu.sync_copy(row_hbm.at[pl.ds(c * ET + off, CHUNK)], rb)
        pltpu.sync_copy(w_hbm.at[pl.ds(off, CHUNK)], sb)
        pltpu.async_copy(tab_hbm.at[gb], rows, sem).wait()

        def _scale(j, carry2):
            w = plsc.load_gather(sb, [jnp.full((LANES,), j, jnp.int32)])
            for q in range(D // LANES):
                rows[j, pl.ds(q * LANES, LANES)] = (
                    rows[j, pl.ds(q * LANES, LANES)] * w)
            return carry2

        lax.fori_loop(0, CHUNK, _scale, 0)
        pltpu.sync_copy(rows, acc.at[rb], add=True)
        return carry

    lax.fori_loop(0, EPT // CHUNK, _chunk, 0)
    plsc.subcore_barrier()
    pltpu.sync_copy(acc.at[pl.ds(s * RPT, RPT)], s_hbm.at[c, pl.ds(s * RPT, RPT)])


def _msg_call(tab_flat, gs_x, rowcat, w_full):
    return pl.kernel(
        _msg_body,
        out_type=jax.ShapeDtypeStruct((NC, SPAD, D), jnp.float32),
        mesh=_sc_mesh(),
        scratch_types=[
            pltpu.VMEM((CHUNK,), jnp.int32),
            pltpu.VMEM((CHUNK,), jnp.int32),
            pltpu.VMEM((CHUNK,), jnp.float32),
            pltpu.VMEM((CHUNK, D), jnp.float32),
            pltpu.VMEM_SHARED((SPAD, D), jnp.float32),
            pltpu.SemaphoreType.DMA,
        ],
        **_SC_PARAMS,
    )(tab_flat, gs_x, rowcat, w_full)


def _tc1_body(x_ref, pw_ref, pb_ref, bg_ref, bb_ref, parts_ref,
              src_ref, dst_ref, et_ref, cw_ref,
              h_ref, dinv_ref, gs_ref, gd_ref, row_ref, tab_ref):
    z = jnp.dot(x_ref[...], pw_ref[...],
                preferred_element_type=jnp.float32) + pb_ref[0][None, :]
    mu = jnp.mean(z, axis=0, keepdims=True)
    zc = z - mu
    var = jnp.mean(zc * zc, axis=0, keepdims=True)
    h = bg_ref[0][None, :] * zc / jnp.sqrt(var + EPS) + bb_ref[0][None, :]
    h = jnp.maximum(h, 0.0)
    h_ref[...] = h
    for r in range(NREL):
        tab_ref[r] = jnp.dot(h, cw_ref[r], preferred_element_type=jnp.float32)
    deg = jnp.sum(parts_ref[...], axis=0, keepdims=True)
    dinv_ref[...] = lax.rsqrt(deg)
    sv = src_ref[...]
    dv = dst_ref[...]
    tn = et_ref[...] * N
    gs_ref[...] = tn + sv
    gd_ref[...] = tn + dv
    th = et_ref[...] * HALF
    row_ref[0] = jnp.where(dv < HALF, th + dv, DUMMY)
    row_ref[1] = jnp.where(dv >= HALF, th + (dv - HALF), DUMMY)


def _comb_body(h_ref, s3_ref, skw_ref, skb_ref, cb_ref, o_ref):
    out = jnp.dot(h_ref[...], skw_ref[...],
                  preferred_element_type=jnp.float32) + skb_ref[0][None, :]
    for r in range(NREL):
        out = out + s3_ref[r] + cb_ref[r][None, :]
    o_ref[...] = out


def _bn_relu_body(z_ref, bg_ref, bb_ref, o_ref):
    z = z_ref[...]
    mu = jnp.mean(z, axis=0, keepdims=True)
    zc = z - mu
    var = jnp.mean(zc * zc, axis=0, keepdims=True)
    o = bg_ref[0][None, :] * zc / jnp.sqrt(var + EPS) + bb_ref[0][None, :]
    o_ref[...] = jnp.maximum(o, 0.0)


def _bn_relu_tab_body(z_ref, bg_ref, bb_ref, cw_ref, o_ref, tab_ref):
    z = z_ref[...]
    mu = jnp.mean(z, axis=0, keepdims=True)
    zc = z - mu
    var = jnp.mean(zc * zc, axis=0, keepdims=True)
    o = bg_ref[0][None, :] * zc / jnp.sqrt(var + EPS) + bb_ref[0][None, :]
    o = jnp.maximum(o, 0.0)
    o_ref[...] = o
    for r in range(NREL):
        tab_ref[r] = jnp.dot(o, cw_ref[r], preferred_element_type=jnp.float32)


def _layer_call(h, s_acc, skw, skb, cb, bng, bnb, cw_next):
    s3 = jnp.stack(
        [jnp.concatenate([s_acc[0, r * HALF:(r + 1) * HALF, :],
                          s_acc[1, r * HALF:(r + 1) * HALF, :]], axis=0)
         for r in range(NREL)])
    z = pl.pallas_call(
        _comb_body,
        out_shape=jax.ShapeDtypeStruct((N, D), jnp.float32),
    )(h, s3, skw, skb, cb)
    if cw_next is None:
        h2 = pl.pallas_call(
            _bn_relu_body,
            out_shape=jax.ShapeDtypeStruct((N, D), jnp.float32),
        )(z, bng, bnb)
        return h2, None
    h2, tab = pl.pallas_call(
        _bn_relu_tab_body,
        out_shape=[jax.ShapeDtypeStruct((N, D), jnp.float32),
                   jax.ShapeDtypeStruct((NREL, N, D), jnp.float32)],
    )(z, bng, bnb, cw_next)
    return h2, tab


def _tc3_body(h_ref, b_ref, w1_ref, b1_ref, g1_ref, c1_ref,
              w2_ref, b2_ref, g2_ref, c2_ref, w3_ref, b3_ref, o_ref):
    gi = lax.broadcasted_iota(jnp.int32, (N, NG), 1)
    onehot = (b_ref[...] == gi).astype(jnp.float32)
    pooled = lax.dot_general(onehot, h_ref[...], (((0,), (0,)), ((), ())),
                             preferred_element_type=jnp.float32, precision=lax.Precision.HIGHEST)

    def bn(t, g_ref, c_ref):
        mu = jnp.mean(t, axis=0, keepdims=True)
        tc = t - mu
        var = jnp.mean(tc * tc, axis=0, keepdims=True)
        return g_ref[0][None, :] * tc / jnp.sqrt(var + EPS) + c_ref[0][None, :]

    z1 = jnp.dot(pooled, w1_ref[...],
                 preferred_element_type=jnp.float32) + b1_ref[0][None, :]
    z1 = jnp.maximum(bn(z1, g1_ref, c1_ref), 0.0)
    z2 = jnp.dot(z1, w2_ref[...],
                 preferred_element_type=jnp.float32) + b2_ref[0][None, :]
    z2 = jnp.maximum(bn(z2, g2_ref, c2_ref), 0.0)
    o_ref[...] = jnp.dot(z2, w3_ref[...],
                         preferred_element_type=jnp.float32) + b3_ref[0][None, :]


def kernel(x, edge_index, edge_type, edge_score, batch,
           pre_W, pre_b, pre_bn_g, pre_bn_b,
           conv_W, conv_b, skip_W, skip_b, layer_bn_g, layer_bn_b,
           post_W1, post_b1, post_bn1_g, post_bn1_b,
           post_W2, post_b2, post_bn2_g, post_bn2_b,
           post_W3, post_b3):
    i32 = jnp.int32
    f32 = jnp.float32
    src = edge_index[0].astype(i32)
    dst = edge_index[1].astype(i32)
    et = edge_type.astype(i32)
    es = edge_score.astype(f32)

    loop = jnp.arange(N, dtype=i32)
    padi = jnp.zeros((PADE,), i32)
    src_x = jnp.concatenate([src, loop, loop, loop, padi])
    dst_x = jnp.concatenate([dst, loop, loop, loop, padi])
    et_x = jnp.concatenate([et, jnp.zeros((N,), i32), jnp.ones((N,), i32),
                            jnp.full((N,), 2, i32), padi])
    es_x = jnp.concatenate([es, jnp.ones((NREL * N,), f32),
                            jnp.zeros((PADE,), f32)])

    parts = _deg_call(dst_x, et_x, es_x)

    h, dinv, gs, gd, rowcat, tab = pl.pallas_call(
        _tc1_body,
        out_shape=[jax.ShapeDtypeStruct((N, D), f32),
                   jax.ShapeDtypeStruct((1, NREL * N), f32),
                   jax.ShapeDtypeStruct((ETR, 128), i32),
                   jax.ShapeDtypeStruct((ETR, 128), i32),
                   jax.ShapeDtypeStruct((2, ETR, 128), i32),
                   jax.ShapeDtypeStruct((NREL, N, D), f32)],
    )(x, pre_W, pre_b.reshape(1, D), pre_bn_g.reshape(1, D),
      pre_bn_b.reshape(1, D), parts,
      src_x.reshape(ETR, 128), dst_x.reshape(ETR, 128), et_x.reshape(ETR, 128),
      conv_W[0])

    gs_flat = gs.reshape(ET)
    row_flat = rowcat.reshape(2 * ET)
    w_full = _w_call(dinv.reshape(NREL * N), gs_flat, gd.reshape(ET), es_x)

    for l in range(2):
        s_acc = _msg_call(tab.reshape(NREL * N, D), gs_flat, row_flat, w_full)
        cw_next = conv_W[1] if l == 0 else None
        h, tab = _layer_call(h, s_acc, skip_W[l], skip_b[l].reshape(1, D),
                             conv_b[l],
                             layer_bn_g[l].reshape(1, D),
                             layer_bn_b[l].reshape(1, D), cw_next)

    out = pl.pallas_call(
        _tc3_body,
        out_shape=jax.ShapeDtypeStruct((NG, 16), f32),
    )(h, batch.astype(i32).reshape(N, 1),
      post_W1, post_b1.reshape(1, D), post_bn1_g.reshape(1, D),
      post_bn1_b.reshape(1, D),
      post_W2, post_b2.reshape(1, 32), post_bn2_g.reshape(1, 32),
      post_bn2_b.reshape(1, 32),
      post_W3, post_b3.reshape(1, 16))
    return out


# speedup vs baseline: 15.3484x; 1.6792x over previous
import jax
import jax.numpy as jnp
from jax import lax
from jax.experimental import pallas as pl
from jax.experimental.pallas import tpu as pltpu
from jax.experimental.pallas import tpu_sc as plsc

N = 10000
E = 320000
NREL = 3
D = 128
NG = 64
EPS = 1e-5

NC = 2
NS = 16
LANES = 16
HALF = N // NC
DUMMY = NREL * HALF
SPAD = 15104
RPT = SPAD // NS
CHUNK = 64
ET = 353280
PADE = ET - E - NREL * N
ETR = ET // 128
EPT = ET // NS
EPW = ET // (NC * NS)

_SC_PARAMS = dict(
    compiler_params=pltpu.CompilerParams(needs_layout_passes=False),
)


def _sc_mesh():
    return plsc.VectorSubcoreMesh(core_axis_name="c", subcore_axis_name="s")


def _deg_body(dst_hbm, et_hbm, es_hbm, part_hbm, dstb, etb, esb, accb):
    c = lax.axis_index("c")
    s = lax.axis_index("s")
    wid = c * NS + s
    base = wid * EPW
    pltpu.sync_copy(dst_hbm.at[pl.ds(base, EPW)], dstb)
    pltpu.sync_copy(et_hbm.at[pl.ds(base, EPW)], etb)
    pltpu.sync_copy(es_hbm.at[pl.ds(base, EPW)], esb)
    zv = jnp.zeros((LANES,), jnp.float32)

    def _zero(i, carry):
        accb[pl.ds(i * LANES, LANES)] = zv
        return carry

    lax.fori_loop(0, NREL * N // LANES, _zero, 0)

    def _edge(i, carry):
        t = etb[pl.ds(i * LANES, LANES)]
        d = dstb[pl.ds(i * LANES, LANES)]
        w = esb[pl.ds(i * LANES, LANES)]
        plsc.addupdate_scatter(accb, [t * N + d], w)
        return carry

    lax.fori_loop(0, EPW // LANES, _edge, 0)
    pltpu.sync_copy(accb, part_hbm.at[wid])


def _deg_call(dst_x, et_x, es_x):
    return pl.kernel(
        _deg_body,
        out_type=jax.ShapeDtypeStruct((NC * NS, NREL * N), jnp.float32),
        mesh=_sc_mesh(),
        scratch_types=[
            pltpu.VMEM((EPW,), jnp.int32),
            pltpu.VMEM((EPW,), jnp.int32),
            pltpu.VMEM((EPW,), jnp.float32),
            pltpu.VMEM((NREL * N,), jnp.float32),
        ],
        **_SC_PARAMS,
    )(dst_x, et_x, es_x)


PBLK = 5520


def _prep_body(dinv_hbm, gs_hbm, gd_hbm, es_hbm, row_hbm, pidx_hbm, w_hbm,
               cnt_hbm, dinvb, gsb, gdb, esb, rwb, pout, wout, cntb):
    c = lax.axis_index("c")
    s = lax.axis_index("s")
    wbase = s * EPT
    pltpu.sync_copy(dinv_hbm, dinvb)

    def _block(bi, o):
        bbase = wbase + bi * PBLK
        pltpu.sync_copy(gs_hbm.at[pl.ds(bbase, PBLK)], gsb)
        pltpu.sync_copy(gd_hbm.at[pl.ds(bbase, PBLK)], gdb)
        pltpu.sync_copy(es_hbm.at[pl.ds(bbase, PBLK)], esb)
        pltpu.sync_copy(row_hbm.at[pl.ds(c * ET + bbase, PBLK)], rwb)

        def _vec(i, o2):
            q = i * LANES
            gsv = gsb[pl.ds(q, LANES)]
            gdv = gdb[pl.ds(q, LANES)]
            esv = esb[pl.ds(q, LANES)]
            rwv = rwb[pl.ds(q, LANES)]
            w = (esv * plsc.load_gather(dinvb, [gsv])
                 * plsc.load_gather(dinvb, [gdv]))
            packed = jnp.bitwise_or(jnp.left_shift(gsv, 14), rwv)
            mask = rwv != DUMMY
            plsc.store_compressed(pout.at[pl.ds(o2, LANES)], packed, mask=mask)
            plsc.store_compressed(wout.at[pl.ds(o2, LANES)], w, mask=mask)
            cntv = plsc.all_reduce_population_count(mask)
            return o2 + cntv[0]

        return lax.fori_loop(0, PBLK // LANES, _vec, o)

    n = lax.fori_loop(0, EPT // PBLK, _block, jnp.int32(0))

    padv = jnp.full((LANES,), DUMMY, jnp.int32)
    zv = jnp.zeros((LANES,), jnp.float32)
    npad = (CHUNK - 1) - ((n + CHUNK - 1) % CHUNK)

    def _pad(i, carry):
        @pl.when(i * LANES < npad)
        def _():
            pout[pl.ds(n + i * LANES, LANES)] = padv
            wout[pl.ds(n + i * LANES, LANES)] = zv
        return carry

    lax.fori_loop(0, CHUNK // LANES, _pad, 0)

    cntb[pl.ds(0, LANES)] = jnp.full((LANES,), n, jnp.int32)
    base = (c * NS + s)
    pltpu.sync_copy(pout.at[pl.ds(0, EPT)], pidx_hbm.at[pl.ds(c * ET + wbase, EPT)])
    pltpu.sync_copy(wout.at[pl.ds(0, EPT)], w_hbm.at[pl.ds(c * ET + wbase, EPT)])
    pltpu.sync_copy(cntb, cnt_hbm.at[pl.ds(base * LANES, LANES)])


def _prep_call(dinv_flat, gs_x, gd_x, es_x, row_flat):
    return pl.kernel(
        _prep_body,
        out_type=[jax.ShapeDtypeStruct((NC * ET,), jnp.int32),
                  jax.ShapeDtypeStruct((NC * ET,), jnp.float32),
                  jax.ShapeDtypeStruct((NC * NS * LANES,), jnp.int32)],
        mesh=_sc_mesh(),
        scratch_types=[
            pltpu.VMEM((NREL * N,), jnp.float32),
            pltpu.VMEM((PBLK,), jnp.int32),
            pltpu.VMEM((PBLK,), jnp.int32),
            pltpu.VMEM((PBLK,), jnp.float32),
            pltpu.VMEM((PBLK,), jnp.int32),
            pltpu.VMEM((EPT + CHUNK,), jnp.int32),
            pltpu.VMEM((EPT + CHUNK,), jnp.float32),
            pltpu.VMEM((LANES,), jnp.int32),
        ],
        **_SC_PARAMS,
    )(dinv_flat, gs_x, gd_x, es_x, row_flat)


def _msg_body(tab_hbm, pidx_hbm, w_hbm, cnt_hbm, s_hbm,
              pb, gb, rb, sb, cb16, rows, acc, sem):
    c = lax.axis_index("c")
    s = lax.axis_index("s")
    zv = jnp.zeros((LANES,), jnp.float32)

    def _zrow(j, carry):
        for q in range(D // LANES):
            rows[j, pl.ds(q * LANES, LANES)] = zv
        return carry

    lax.fori_loop(0, CHUNK, _zrow, 0)

    def _zstripe(i, carry):
        pltpu.sync_copy(rows, acc.at[pl.ds(s * RPT + i * CHUNK, CHUNK)])
        return carry

    lax.fori_loop(0, RPT // CHUNK, _zstripe, 0)
    pltpu.sync_copy(rows.at[pl.ds(0, RPT % CHUNK)],
                    acc.at[pl.ds(s * RPT + (RPT // CHUNK) * CHUNK, RPT % CHUNK)])
    plsc.subcore_barrier()

    pltpu.sync_copy(cnt_hbm.at[pl.ds((c * NS + s) * LANES, LANES)], cb16)
    n = cb16[pl.ds(0, LANES)][0]
    nch = (n + (CHUNK - 1)) // CHUNK

    def _chunk(ci, carry):
        off = c * ET + s * EPT + ci * CHUNK
        pltpu.sync_copy(pidx_hbm.at[pl.ds(off, CHUNK)], pb)
        pltpu.sync_copy(w_hbm.at[pl.ds(off, CHUNK)], sb)

        def _unpack(j, carry2):
            q = j * LANES
            pv = pb[pl.ds(q, LANES)]
            gb[pl.ds(q, LANES)] = jnp.right_shift(pv, 14)
            rb[pl.ds(q, LANES)] = jnp.bitwise_and(pv, 16383)
            return carry2

        lax.fori_loop(0, CHUNK // LANES, _unpack, 0)
        pltpu.async_copy(tab_hbm.at[gb], rows, sem).wait()

        def _scale(j, carry2):
            w = plsc.load_gather(sb, [jnp.full((LANES,), j, jnp.int32)])
            for q in range(D // LANES):
                rows[j, pl.ds(q * LANES, LANES)] = (
                    rows[j, pl.ds(q * LANES, LANES)] * w)
            return carry2

        lax.fori_loop(0, CHUNK, _scale, 0)
        pltpu.sync_copy(rows, acc.at[rb], add=True)
        return carry

    lax.fori_loop(0, nch, _chunk, 0)
    plsc.subcore_barrier()
    pltpu.sync_copy(acc.at[pl.ds(s * RPT, RPT)], s_hbm.at[c, pl.ds(s * RPT, RPT)])


def _msg_call(tab_flat, pidx, w_c, cnt):
    return pl.kernel(
        _msg_body,
        out_type=jax.ShapeDtypeStruct((NC, SPAD, D), jnp.float32),
        mesh=_sc_mesh(),
        scratch_types=[
            pltpu.VMEM((CHUNK,), jnp.int32),
            pltpu.VMEM((CHUNK,), jnp.int32),
            pltpu.VMEM((CHUNK,), jnp.int32),
            pltpu.VMEM((CHUNK,), jnp.float32),
            pltpu.VMEM((LANES,), jnp.int32),
            pltpu.VMEM((CHUNK, D), jnp.float32),
            pltpu.VMEM_SHARED((SPAD, D), jnp.float32),
            pltpu.SemaphoreType.DMA,
        ],
        **_SC_PARAMS,
    )(tab_flat, pidx, w_c, cnt)


def _tc1_body(x_ref, pw_ref, pb_ref, bg_ref, bb_ref, parts_ref,
              src_ref, dst_ref, et_ref, cw_ref,
              h_ref, dinv_ref, gs_ref, gd_ref, row_ref, tab_ref):
    z = jnp.dot(x_ref[...], pw_ref[...],
                preferred_element_type=jnp.float32) + pb_ref[0][None, :]
    mu = jnp.mean(z, axis=0, keepdims=True)
    zc = z - mu
    var = jnp.mean(zc * zc, axis=0, keepdims=True)
    h = bg_ref[0][None, :] * zc / jnp.sqrt(var + EPS) + bb_ref[0][None, :]
    h = jnp.maximum(h, 0.0)
    h_ref[...] = h
    for r in range(NREL):
        tab_ref[r] = jnp.dot(h, cw_ref[r], preferred_element_type=jnp.float32)
    deg = jnp.sum(parts_ref[...], axis=0, keepdims=True)
    dinv_ref[...] = lax.rsqrt(deg)
    sv = src_ref[...]
    dv = dst_ref[...]
    tn = et_ref[...] * N
    gs_ref[...] = tn + sv
    gd_ref[...] = tn + dv
    th = et_ref[...] * HALF
    row_ref[0] = jnp.where(dv < HALF, th + dv, DUMMY)
    row_ref[1] = jnp.where(dv >= HALF, th + (dv - HALF), DUMMY)


def _comb_body(h_ref, s3_ref, skw_ref, skb_ref, cb_ref, o_ref):
    out = jnp.dot(h_ref[...], skw_ref[...],
                  preferred_element_type=jnp.float32) + skb_ref[0][None, :]
    for r in range(NREL):
        out = out + s3_ref[r] + cb_ref[r][None, :]
    o_ref[...] = out


def _bn_relu_body(z_ref, bg_ref, bb_ref, o_ref):
    z = z_ref[...]
    mu = jnp.mean(z, axis=0, keepdims=True)
    zc = z - mu
    var = jnp.mean(zc * zc, axis=0, keepdims=True)
    o = bg_ref[0][None, :] * zc / jnp.sqrt(var + EPS) + bb_ref[0][None, :]
    o_ref[...] = jnp.maximum(o, 0.0)


def _bn_relu_tab_body(z_ref, bg_ref, bb_ref, cw_ref, o_ref, tab_ref):
    z = z_ref[...]
    mu = jnp.mean(z, axis=0, keepdims=True)
    zc = z - mu
    var = jnp.mean(zc * zc, axis=0, keepdims=True)
    o = bg_ref[0][None, :] * zc / jnp.sqrt(var + EPS) + bb_ref[0][None, :]
    o = jnp.maximum(o, 0.0)
    o_ref[...] = o
    for r in range(NREL):
        tab_ref[r] = jnp.dot(o, cw_ref[r], preferred_element_type=jnp.float32)


def _layer_call(h, s_acc, skw, skb, cb, bng, bnb, cw_next):
    s3 = jnp.stack(
        [jnp.concatenate([s_acc[0, r * HALF:(r + 1) * HALF, :],
                          s_acc[1, r * HALF:(r + 1) * HALF, :]], axis=0)
         for r in range(NREL)])
    z = pl.pallas_call(
        _comb_body,
        out_shape=jax.ShapeDtypeStruct((N, D), jnp.float32),
    )(h, s3, skw, skb, cb)
    if cw_next is None:
        h2 = pl.pallas_call(
            _bn_relu_body,
            out_shape=jax.ShapeDtypeStruct((N, D), jnp.float32),
        )(z, bng, bnb)
        return h2, None
    h2, tab = pl.pallas_call(
        _bn_relu_tab_body,
        out_shape=[jax.ShapeDtypeStruct((N, D), jnp.float32),
                   jax.ShapeDtypeStruct((NREL, N, D), jnp.float32)],
    )(z, bng, bnb, cw_next)
    return h2, tab


def _tc3_body(h_ref, b_ref, w1_ref, b1_ref, g1_ref, c1_ref,
              w2_ref, b2_ref, g2_ref, c2_ref, w3_ref, b3_ref, o_ref):
    gi = lax.broadcasted_iota(jnp.int32, (N, NG), 1)
    onehot = (b_ref[...] == gi).astype(jnp.float32)
    pooled = lax.dot_general(onehot, h_ref[...], (((0,), (0,)), ((), ())),
                             preferred_element_type=jnp.float32, precision=lax.Precision.HIGHEST)

    def bn(t, g_ref, c_ref):
        mu = jnp.mean(t, axis=0, keepdims=True)
        tc = t - mu
        var = jnp.mean(tc * tc, axis=0, keepdims=True)
        return g_ref[0][None, :] * tc / jnp.sqrt(var + EPS) + c_ref[0][None, :]

    z1 = jnp.dot(pooled, w1_ref[...],
                 preferred_element_type=jnp.float32) + b1_ref[0][None, :]
    z1 = jnp.maximum(bn(z1, g1_ref, c1_ref), 0.0)
    z2 = jnp.dot(z1, w2_ref[...],
                 preferred_element_type=jnp.float32) + b2_ref[0][None, :]
    z2 = jnp.maximum(bn(z2, g2_ref, c2_ref), 0.0)
    o_ref[...] = jnp.dot(z2, w3_ref[...],
                         preferred_element_type=jnp.float32) + b3_ref[0][None, :]


def kernel(x, edge_index, edge_type, edge_score, batch,
           pre_W, pre_b, pre_bn_g, pre_bn_b,
           conv_W, conv_b, skip_W, skip_b, layer_bn_g, layer_bn_b,
           post_W1, post_b1, post_bn1_g, post_bn1_b,
           post_W2, post_b2, post_bn2_g, post_bn2_b,
           post_W3, post_b3):
    i32 = jnp.int32
    f32 = jnp.float32
    src = edge_index[0].astype(i32)
    dst = edge_index[1].astype(i32)
    et = edge_type.astype(i32)
    es = edge_score.astype(f32)

    loop = jnp.arange(N, dtype=i32)
    padi = jnp.zeros((PADE,), i32)
    src_x = jnp.concatenate([src, loop, loop, loop, padi])
    dst_x = jnp.concatenate([dst, loop, loop, loop, padi])
    et_x = jnp.concatenate([et, jnp.zeros((N,), i32), jnp.ones((N,), i32),
                            jnp.full((N,), 2, i32), padi])
    es_x = jnp.concatenate([es, jnp.ones((NREL * N,), f32),
                            jnp.zeros((PADE,), f32)])

    parts = _deg_call(dst_x, et_x, es_x)

    h, dinv, gs, gd, rowcat, tab = pl.pallas_call(
        _tc1_body,
        out_shape=[jax.ShapeDtypeStruct((N, D), f32),
                   jax.ShapeDtypeStruct((1, NREL * N), f32),
                   jax.ShapeDtypeStruct((ETR, 128), i32),
                   jax.ShapeDtypeStruct((ETR, 128), i32),
                   jax.ShapeDtypeStruct((2, ETR, 128), i32),
                   jax.ShapeDtypeStruct((NREL, N, D), f32)],
    )(x, pre_W, pre_b.reshape(1, D), pre_bn_g.reshape(1, D),
      pre_bn_b.reshape(1, D), parts,
      src_x.reshape(ETR, 128), dst_x.reshape(ETR, 128), et_x.reshape(ETR, 128),
      conv_W[0])

    gs_flat = gs.reshape(ET)
    row_flat = rowcat.reshape(2 * ET)
    pidx, w_c, cnt = _prep_call(dinv.reshape(NREL * N), gs_flat,
                                gd.reshape(ET), es_x, row_flat)

    for l in range(2):
        s_acc = _msg_call(tab.reshape(NREL * N, D), pidx, w_c, cnt)
        cw_next = conv_W[1] if l == 0 else None
        h, tab = _layer_call(h, s_acc, skip_W[l], skip_b[l].reshape(1, D),
                             conv_b[l],
                             layer_bn_g[l].reshape(1, D),
                             layer_bn_b[l].reshape(1, D), cw_next)

    out = pl.pallas_call(
        _tc3_body,
        out_shape=jax.ShapeDtypeStruct((NG, 16), f32),
    )(h, batch.astype(i32).reshape(N, 1),
      post_W1, post_b1.reshape(1, D), post_bn1_g.reshape(1, D),
      post_bn1_b.reshape(1, D),
      post_W2, post_b2.reshape(1, 32), post_bn2_g.reshape(1, 32),
      post_bn2_b.reshape(1, 32),
      post_W3, post_b3.reshape(1, 16))
    return out


# speedup vs baseline: 17.9555x; 1.1699x over previous
import jax
import jax.numpy as jnp
from jax import lax
from jax.experimental import pallas as pl
from jax.experimental.pallas import tpu as pltpu
from jax.experimental.pallas import tpu_sc as plsc

N = 10000
E = 320000
NREL = 3
D = 128
NG = 64
EPS = 1e-5

NC = 2
NS = 16
LANES = 16
HALF = N // NC
DUMMY = NREL * HALF
SPAD = 15104
RPT = SPAD // NS
CHUNK = 32
ET = 353280
PADE = ET - E - NREL * N
ETR = ET // 128
EPT = ET // NS
EPW = ET // (NC * NS)

_SC_PARAMS = dict(
    compiler_params=pltpu.CompilerParams(needs_layout_passes=False),
)


def _sc_mesh():
    return plsc.VectorSubcoreMesh(core_axis_name="c", subcore_axis_name="s")


def _deg_body(dst_hbm, et_hbm, es_hbm, part_hbm, dstb, etb, esb, accb):
    c = lax.axis_index("c")
    s = lax.axis_index("s")
    wid = c * NS + s
    base = wid * EPW
    pltpu.sync_copy(dst_hbm.at[pl.ds(base, EPW)], dstb)
    pltpu.sync_copy(et_hbm.at[pl.ds(base, EPW)], etb)
    pltpu.sync_copy(es_hbm.at[pl.ds(base, EPW)], esb)
    zv = jnp.zeros((LANES,), jnp.float32)

    def _zero(i, carry):
        accb[pl.ds(i * LANES, LANES)] = zv
        return carry

    lax.fori_loop(0, NREL * N // LANES, _zero, 0)

    def _edge(i, carry):
        t = etb[pl.ds(i * LANES, LANES)]
        d = dstb[pl.ds(i * LANES, LANES)]
        w = esb[pl.ds(i * LANES, LANES)]
        plsc.addupdate_scatter(accb, [t * N + d], w)
        return carry

    lax.fori_loop(0, EPW // LANES, _edge, 0)
    pltpu.sync_copy(accb, part_hbm.at[wid])


def _deg_call(dst_x, et_x, es_x):
    return pl.kernel(
        _deg_body,
        out_type=jax.ShapeDtypeStruct((NC * NS, NREL * N), jnp.float32),
        mesh=_sc_mesh(),
        scratch_types=[
            pltpu.VMEM((EPW,), jnp.int32),
            pltpu.VMEM((EPW,), jnp.int32),
            pltpu.VMEM((EPW,), jnp.float32),
            pltpu.VMEM((NREL * N,), jnp.float32),
        ],
        **_SC_PARAMS,
    )(dst_x, et_x, es_x)


PBLK = 5520


def _prep_body(dinv_hbm, gs_hbm, gd_hbm, es_hbm, row_hbm, pidx_hbm, w_hbm,
               cnt_hbm, dinvb, gsb, gdb, esb, rwb, pout, wout, cntb):
    c = lax.axis_index("c")
    s = lax.axis_index("s")
    wbase = s * EPT
    pltpu.sync_copy(dinv_hbm, dinvb)

    def _block(bi, o):
        bbase = wbase + bi * PBLK
        pltpu.sync_copy(gs_hbm.at[pl.ds(bbase, PBLK)], gsb)
        pltpu.sync_copy(gd_hbm.at[pl.ds(bbase, PBLK)], gdb)
        pltpu.sync_copy(es_hbm.at[pl.ds(bbase, PBLK)], esb)
        pltpu.sync_copy(row_hbm.at[pl.ds(c * ET + bbase, PBLK)], rwb)

        def _vec(i, o2):
            q = i * LANES
            gsv = gsb[pl.ds(q, LANES)]
            gdv = gdb[pl.ds(q, LANES)]
            esv = esb[pl.ds(q, LANES)]
            rwv = rwb[pl.ds(q, LANES)]
            w = (esv * plsc.load_gather(dinvb, [gsv])
                 * plsc.load_gather(dinvb, [gdv]))
            packed = jnp.bitwise_or(jnp.left_shift(gsv, 14), rwv)
            mask = rwv != DUMMY
            plsc.store_compressed(pout.at[pl.ds(o2, LANES)], packed, mask=mask)
            plsc.store_compressed(wout.at[pl.ds(o2, LANES)], w, mask=mask)
            cntv = plsc.all_reduce_population_count(mask)
            return o2 + cntv[0]

        return lax.fori_loop(0, PBLK // LANES, _vec, o)

    n = lax.fori_loop(0, EPT // PBLK, _block, jnp.int32(0))

    padv = jnp.full((LANES,), DUMMY, jnp.int32)
    zv = jnp.zeros((LANES,), jnp.float32)
    npad = (CHUNK - 1) - ((n + CHUNK - 1) % CHUNK)

    def _pad(i, carry):
        @pl.when(i * LANES < npad)
        def _():
            pout[pl.ds(n + i * LANES, LANES)] = padv
            wout[pl.ds(n + i * LANES, LANES)] = zv
        return carry

    lax.fori_loop(0, CHUNK // LANES, _pad, 0)

    cntb[pl.ds(0, LANES)] = jnp.full((LANES,), n, jnp.int32)
    base = (c * NS + s)
    pltpu.sync_copy(pout.at[pl.ds(0, EPT)], pidx_hbm.at[pl.ds(c * ET + wbase, EPT)])
    pltpu.sync_copy(wout.at[pl.ds(0, EPT)], w_hbm.at[pl.ds(c * ET + wbase, EPT)])
    pltpu.sync_copy(cntb, cnt_hbm.at[pl.ds(base * LANES, LANES)])


def _prep_call(dinv_flat, gs_x, gd_x, es_x, row_flat):
    return pl.kernel(
        _prep_body,
        out_type=[jax.ShapeDtypeStruct((NC * ET,), jnp.int32),
                  jax.ShapeDtypeStruct((NC * ET,), jnp.float32),
                  jax.ShapeDtypeStruct((NC * NS * LANES,), jnp.int32)],
        mesh=_sc_mesh(),
        scratch_types=[
            pltpu.VMEM((NREL * N,), jnp.float32),
            pltpu.VMEM((PBLK,), jnp.int32),
            pltpu.VMEM((PBLK,), jnp.int32),
            pltpu.VMEM((PBLK,), jnp.float32),
            pltpu.VMEM((PBLK,), jnp.int32),
            pltpu.VMEM((EPT + CHUNK,), jnp.int32),
            pltpu.VMEM((EPT + CHUNK,), jnp.float32),
            pltpu.VMEM((LANES,), jnp.int32),
        ],
        **_SC_PARAMS,
    )(dinv_flat, gs_x, gd_x, es_x, row_flat)


def _msg_body(tab_hbm, pidx_hbm, w_hbm, cnt_hbm, s_hbm,
              pb, gb2, rb2, sb2, cb16, rows, acc, sem):
    c = lax.axis_index("c")
    s = lax.axis_index("s")
    zv = jnp.zeros((LANES,), jnp.float32)

    def _zrow(j, carry):
        for q in range(D // LANES):
            rows[0, j, pl.ds(q * LANES, LANES)] = zv
        return carry

    lax.fori_loop(0, CHUNK, _zrow, 0)

    def _zstripe(i, carry):
        pltpu.sync_copy(rows.at[0], acc.at[pl.ds(s * RPT + i * CHUNK, CHUNK)])
        return carry

    lax.fori_loop(0, RPT // CHUNK, _zstripe, 0)
    if RPT % CHUNK:
        pltpu.sync_copy(
            rows.at[0, pl.ds(0, RPT % CHUNK)],
            acc.at[pl.ds(s * RPT + (RPT // CHUNK) * CHUNK, RPT % CHUNK)])
    plsc.subcore_barrier()

    pltpu.sync_copy(cnt_hbm.at[pl.ds((c * NS + s) * LANES, LANES)], cb16)
    n = cb16[pl.ds(0, LANES)][0]
    nch = (n + (CHUNK - 1)) // CHUNK
    base = c * ET + s * EPT

    def _issue(ci, b):
        off = base + ci * CHUNK
        pltpu.sync_copy(pidx_hbm.at[pl.ds(off, CHUNK)], pb)
        pltpu.sync_copy(w_hbm.at[pl.ds(off, CHUNK)],
                        sb2.at[b])

        def _unpack(j, carry):
            q = j * LANES
            pv = pb[pl.ds(q, LANES)]
            gb2[b, pl.ds(q, LANES)] = jnp.right_shift(pv, 14)
            rb2[b, pl.ds(q, LANES)] = jnp.bitwise_and(pv, 16383)
            return carry

        lax.fori_loop(0, CHUNK // LANES, _unpack, 0)
        pltpu.async_copy(tab_hbm.at[gb2.at[b]], rows.at[b], sem.at[b])

    @pl.when(nch > 0)
    def _():
        _issue(jnp.int32(0), jnp.int32(0))

    def _chunk(ci, carry):
        b = lax.rem(ci, 2)

        @pl.when(ci + 1 < nch)
        def _():
            _issue(ci + 1, 1 - b)

        pltpu.make_async_copy(tab_hbm.at[gb2.at[b]], rows.at[b],
                              sem.at[b]).wait()

        def _scale(j, carry2):
            w = plsc.load_gather(
                sb2, [jnp.full((LANES,), b, jnp.int32),
                      jnp.full((LANES,), j, jnp.int32)])
            for q in range(D // LANES):
                rows[b, j, pl.ds(q * LANES, LANES)] = (
                    rows[b, j, pl.ds(q * LANES, LANES)] * w)
            return carry2

        lax.fori_loop(0, CHUNK, _scale, 0)
        pltpu.sync_copy(rows.at[b], acc.at[rb2.at[b]], add=True)
        return carry

    lax.fori_loop(0, nch, _chunk, 0)
    plsc.subcore_barrier()
    pltpu.sync_copy(acc.at[pl.ds(s * RPT, RPT)], s_hbm.at[c, pl.ds(s * RPT, RPT)])


def _msg_call(tab_flat, pidx, w_c, cnt):
    return pl.kernel(
        _msg_body,
        out_type=jax.ShapeDtypeStruct((NC, SPAD, D), jnp.float32),
        mesh=_sc_mesh(),
        scratch_types=[
            pltpu.VMEM((CHUNK,), jnp.int32),
            pltpu.VMEM((2, CHUNK), jnp.int32),
            pltpu.VMEM((2, CHUNK), jnp.int32),
            pltpu.VMEM((2, CHUNK), jnp.float32),
            pltpu.VMEM((LANES,), jnp.int32),
            pltpu.VMEM((2, CHUNK, D), jnp.float32),
            pltpu.VMEM_SHARED((SPAD, D), jnp.float32),
            pltpu.SemaphoreType.DMA((2,)),
        ],
        **_SC_PARAMS,
    )(tab_flat, pidx, w_c, cnt)


def _tc1_body(x_ref, pw_ref, pb_ref, bg_ref, bb_ref, parts_ref,
              src_ref, dst_ref, et_ref, cw_ref,
              h_ref, dinv_ref, gs_ref, gd_ref, row_ref, tab_ref):
    z = jnp.dot(x_ref[...], pw_ref[...],
                preferred_element_type=jnp.float32) + pb_ref[0][None, :]
    mu = jnp.mean(z, axis=0, keepdims=True)
    zc = z - mu
    var = jnp.mean(zc * zc, axis=0, keepdims=True)
    h = bg_ref[0][None, :] * zc / jnp.sqrt(var + EPS) + bb_ref[0][None, :]
    h = jnp.maximum(h, 0.0)
    h_ref[...] = h
    for r in range(NREL):
        tab_ref[r] = jnp.dot(h, cw_ref[r], preferred_element_type=jnp.float32)
    deg = jnp.sum(parts_ref[...], axis=0, keepdims=True)
    dinv_ref[...] = lax.rsqrt(deg)
    sv = src_ref[...]
    dv = dst_ref[...]
    tn = et_ref[...] * N
    gs_ref[...] = tn + sv
    gd_ref[...] = tn + dv
    th = et_ref[...] * HALF
    row_ref[0] = jnp.where(dv < HALF, th + dv, DUMMY)
    row_ref[1] = jnp.where(dv >= HALF, th + (dv - HALF), DUMMY)


def _comb_body(h_ref, s3_ref, skw_ref, skb_ref, cb_ref, o_ref):
    out = jnp.dot(h_ref[...], skw_ref[...],
                  preferred_element_type=jnp.float32) + skb_ref[0][None, :]
    for r in range(NREL):
        out = out + s3_ref[r] + cb_ref[r][None, :]
    o_ref[...] = out


def _bn_relu_body(z_ref, bg_ref, bb_ref, o_ref):
    z = z_ref[...]
    mu = jnp.mean(z, axis=0, keepdims=True)
    zc = z - mu
    var = jnp.mean(zc * zc, axis=0, keepdims=True)
    o = bg_ref[0][None, :] * zc / jnp.sqrt(var + EPS) + bb_ref[0][None, :]
    o_ref[...] = jnp.maximum(o, 0.0)


def _bn_relu_tab_body(z_ref, bg_ref, bb_ref, cw_ref, o_ref, tab_ref):
    z = z_ref[...]
    mu = jnp.mean(z, axis=0, keepdims=True)
    zc = z - mu
    var = jnp.mean(zc * zc, axis=0, keepdims=True)
    o = bg_ref[0][None, :] * zc / jnp.sqrt(var + EPS) + bb_ref[0][None, :]
    o = jnp.maximum(o, 0.0)
    o_ref[...] = o
    for r in range(NREL):
        tab_ref[r] = jnp.dot(o, cw_ref[r], preferred_element_type=jnp.float32)


def _layer_call(h, s_acc, skw, skb, cb, bng, bnb, cw_next):
    s3 = jnp.stack(
        [jnp.concatenate([s_acc[0, r * HALF:(r + 1) * HALF, :],
                          s_acc[1, r * HALF:(r + 1) * HALF, :]], axis=0)
         for r in range(NREL)])
    z = pl.pallas_call(
        _comb_body,
        out_shape=jax.ShapeDtypeStruct((N, D), jnp.float32),
    )(h, s3, skw, skb, cb)
    if cw_next is None:
        h2 = pl.pallas_call(
            _bn_relu_body,
            out_shape=jax.ShapeDtypeStruct((N, D), jnp.float32),
        )(z, bng, bnb)
        return h2, None
    h2, tab = pl.pallas_call(
        _bn_relu_tab_body,
        out_shape=[jax.ShapeDtypeStruct((N, D), jnp.float32),
                   jax.ShapeDtypeStruct((NREL, N, D), jnp.float32)],
    )(z, bng, bnb, cw_next)
    return h2, tab


def _tc3_body(h_ref, b_ref, w1_ref, b1_ref, g1_ref, c1_ref,
              w2_ref, b2_ref, g2_ref, c2_ref, w3_ref, b3_ref, o_ref):
    gi = lax.broadcasted_iota(jnp.int32, (N, NG), 1)
    onehot = (b_ref[...] == gi).astype(jnp.float32)
    pooled = lax.dot_general(onehot, h_ref[...], (((0,), (0,)), ((), ())),
                             preferred_element_type=jnp.float32, precision=lax.Precision.HIGHEST)

    def bn(t, g_ref, c_ref):
        mu = jnp.mean(t, axis=0, keepdims=True)
        tc = t - mu
        var = jnp.mean(tc * tc, axis=0, keepdims=True)
        return g_ref[0][None, :] * tc / jnp.sqrt(var + EPS) + c_ref[0][None, :]

    z1 = jnp.dot(pooled, w1_ref[...],
                 preferred_element_type=jnp.float32) + b1_ref[0][None, :]
    z1 = jnp.maximum(bn(z1, g1_ref, c1_ref), 0.0)
    z2 = jnp.dot(z1, w2_ref[...],
                 preferred_element_type=jnp.float32) + b2_ref[0][None, :]
    z2 = jnp.maximum(bn(z2, g2_ref, c2_ref), 0.0)
    o_ref[...] = jnp.dot(z2, w3_ref[...],
                         preferred_element_type=jnp.float32) + b3_ref[0][None, :]


def kernel(x, edge_index, edge_type, edge_score, batch,
           pre_W, pre_b, pre_bn_g, pre_bn_b,
           conv_W, conv_b, skip_W, skip_b, layer_bn_g, layer_bn_b,
           post_W1, post_b1, post_bn1_g, post_bn1_b,
           post_W2, post_b2, post_bn2_g, post_bn2_b,
           post_W3, post_b3):
    i32 = jnp.int32
    f32 = jnp.float32
    src = edge_index[0].astype(i32)
    dst = edge_index[1].astype(i32)
    et = edge_type.astype(i32)
    es = edge_score.astype(f32)

    loop = jnp.arange(N, dtype=i32)
    padi = jnp.zeros((PADE,), i32)
    src_x = jnp.concatenate([src, loop, loop, loop, padi])
    dst_x = jnp.concatenate([dst, loop, loop, loop, padi])
    et_x = jnp.concatenate([et, jnp.zeros((N,), i32), jnp.ones((N,), i32),
                            jnp.full((N,), 2, i32), padi])
    es_x = jnp.concatenate([es, jnp.ones((NREL * N,), f32),
                            jnp.zeros((PADE,), f32)])

    parts = _deg_call(dst_x, et_x, es_x)

    h, dinv, gs, gd, rowcat, tab = pl.pallas_call(
        _tc1_body,
        out_shape=[jax.ShapeDtypeStruct((N, D), f32),
                   jax.ShapeDtypeStruct((1, NREL * N), f32),
                   jax.ShapeDtypeStruct((ETR, 128), i32),
                   jax.ShapeDtypeStruct((ETR, 128), i32),
                   jax.ShapeDtypeStruct((2, ETR, 128), i32),
                   jax.ShapeDtypeStruct((NREL, N, D), f32)],
    )(x, pre_W, pre_b.reshape(1, D), pre_bn_g.reshape(1, D),
      pre_bn_b.reshape(1, D), parts,
      src_x.reshape(ETR, 128), dst_x.reshape(ETR, 128), et_x.reshape(ETR, 128),
      conv_W[0])

    gs_flat = gs.reshape(ET)
    row_flat = rowcat.reshape(2 * ET)
    pidx, w_c, cnt = _prep_call(dinv.reshape(NREL * N), gs_flat,
                                gd.reshape(ET), es_x, row_flat)

    for l in range(2):
        s_acc = _msg_call(tab.reshape(NREL * N, D), pidx, w_c, cnt)
        cw_next = conv_W[1] if l == 0 else None
        h, tab = _layer_call(h, s_acc, skip_W[l], skip_b[l].reshape(1, D),
                             conv_b[l],
                             layer_bn_g[l].reshape(1, D),
                             layer_bn_b[l].reshape(1, D), cw_next)

    out = pl.pallas_call(
        _tc3_body,
        out_shape=jax.ShapeDtypeStruct((NG, 16), f32),
    )(h, batch.astype(i32).reshape(N, 1),
      post_W1, post_b1.reshape(1, D), post_bn1_g.reshape(1, D),
      post_bn1_b.reshape(1, D),
      post_W2, post_b2.reshape(1, 32), post_bn2_g.reshape(1, 32),
      post_bn2_b.reshape(1, 32),
      post_W3, post_b3.reshape(1, 16))
    return out


# speedup vs baseline: 17.9846x; 1.0016x over previous
import jax
import jax.numpy as jnp
from jax import lax
from jax.experimental import pallas as pl
from jax.experimental.pallas import tpu as pltpu
from jax.experimental.pallas import tpu_sc as plsc

N = 10000
E = 320000
NREL = 3
D = 128
NG = 64
EPS = 1e-5

NC = 2
NS = 16
LANES = 16
HALF = N // NC
DUMMY = NREL * HALF
SPAD = 15104
RPT = SPAD // NS
CHUNK = 32
ET = 353280
PADE = ET - E - NREL * N
ETR = ET // 128
EPT = ET // NS
EPW = ET // (NC * NS)

_SC_PARAMS = dict(
    compiler_params=pltpu.CompilerParams(needs_layout_passes=False),
)


def _sc_mesh():
    return plsc.VectorSubcoreMesh(core_axis_name="c", subcore_axis_name="s")


def _deg_body(dst_hbm, et_hbm, es_hbm, part_hbm, dstb, etb, esb, accb):
    c = lax.axis_index("c")
    s = lax.axis_index("s")
    wid = c * NS + s
    base = wid * EPW
    pltpu.sync_copy(dst_hbm.at[pl.ds(base, EPW)], dstb)
    pltpu.sync_copy(et_hbm.at[pl.ds(base, EPW)], etb)
    pltpu.sync_copy(es_hbm.at[pl.ds(base, EPW)], esb)
    zv = jnp.zeros((LANES,), jnp.float32)

    def _zero(i, carry):
        accb[pl.ds(i * LANES, LANES)] = zv
        return carry

    lax.fori_loop(0, NREL * N // LANES, _zero, 0)

    def _edge(i, carry):
        t = etb[pl.ds(i * LANES, LANES)]
        d = dstb[pl.ds(i * LANES, LANES)]
        w = esb[pl.ds(i * LANES, LANES)]
        plsc.addupdate_scatter(accb, [t * N + d], w)
        return carry

    lax.fori_loop(0, EPW // LANES, _edge, 0)
    pltpu.sync_copy(accb, part_hbm.at[wid])


def _deg_call(dst_x, et_x, es_x):
    return pl.kernel(
        _deg_body,
        out_type=jax.ShapeDtypeStruct((NC * NS, NREL * N), jnp.float32),
        mesh=_sc_mesh(),
        scratch_types=[
            pltpu.VMEM((EPW,), jnp.int32),
            pltpu.VMEM((EPW,), jnp.int32),
            pltpu.VMEM((EPW,), jnp.float32),
            pltpu.VMEM((NREL * N,), jnp.float32),
        ],
        **_SC_PARAMS,
    )(dst_x, et_x, es_x)


PBLK = 5520


def _prep_body(dinv_hbm, gs_hbm, gd_hbm, es_hbm, row_hbm, pidx_hbm, w_hbm,
               cnt_hbm, dinvb, gsb, gdb, esb, rwb, pout, wout, cntb):
    c = lax.axis_index("c")
    s = lax.axis_index("s")
    wbase = s * EPT
    pltpu.sync_copy(dinv_hbm, dinvb)

    def _block(bi, o):
        bbase = wbase + bi * PBLK
        pltpu.sync_copy(gs_hbm.at[pl.ds(bbase, PBLK)], gsb)
        pltpu.sync_copy(gd_hbm.at[pl.ds(bbase, PBLK)], gdb)
        pltpu.sync_copy(es_hbm.at[pl.ds(bbase, PBLK)], esb)
        pltpu.sync_copy(row_hbm.at[pl.ds(c * ET + bbase, PBLK)], rwb)

        def _vec(i, o2):
            q = i * LANES
            gsv = gsb[pl.ds(q, LANES)]
            gdv = gdb[pl.ds(q, LANES)]
            esv = esb[pl.ds(q, LANES)]
            rwv = rwb[pl.ds(q, LANES)]
            w = (esv * plsc.load_gather(dinvb, [gsv])
                 * plsc.load_gather(dinvb, [gdv]))
            packed = jnp.bitwise_or(jnp.left_shift(gsv, 14), rwv)
            mask = rwv != DUMMY
            plsc.store_compressed(pout.at[pl.ds(o2, LANES)], packed, mask=mask)
            plsc.store_compressed(wout.at[pl.ds(o2, LANES)], w, mask=mask)
            cntv = plsc.all_reduce_population_count(mask)
            return o2 + cntv[0]

        return lax.fori_loop(0, PBLK // LANES, _vec, o)

    n = lax.fori_loop(0, EPT // PBLK, _block, jnp.int32(0))

    padv = jnp.full((LANES,), DUMMY, jnp.int32)
    zv = jnp.zeros((LANES,), jnp.float32)
    npad = (CHUNK - 1) - ((n + CHUNK - 1) % CHUNK)

    def _pad(i, carry):
        @pl.when(i * LANES < npad)
        def _():
            pout[pl.ds(n + i * LANES, LANES)] = padv
            wout[pl.ds(n + i * LANES, LANES)] = zv
        return carry

    lax.fori_loop(0, CHUNK // LANES, _pad, 0)

    cntb[pl.ds(0, LANES)] = jnp.full((LANES,), n, jnp.int32)
    base = (c * NS + s)
    pltpu.sync_copy(pout.at[pl.ds(0, EPT)], pidx_hbm.at[pl.ds(c * ET + wbase, EPT)])
    pltpu.sync_copy(wout.at[pl.ds(0, EPT)], w_hbm.at[pl.ds(c * ET + wbase, EPT)])
    pltpu.sync_copy(cntb, cnt_hbm.at[pl.ds(base * LANES, LANES)])


def _prep_call(dinv_flat, gs_x, gd_x, es_x, row_flat):
    return pl.kernel(
        _prep_body,
        out_type=[jax.ShapeDtypeStruct((NC * ET,), jnp.int32),
                  jax.ShapeDtypeStruct((NC * ET,), jnp.float32),
                  jax.ShapeDtypeStruct((NC * NS * LANES,), jnp.int32)],
        mesh=_sc_mesh(),
        scratch_types=[
            pltpu.VMEM((NREL * N,), jnp.float32),
            pltpu.VMEM((PBLK,), jnp.int32),
            pltpu.VMEM((PBLK,), jnp.int32),
            pltpu.VMEM((PBLK,), jnp.float32),
            pltpu.VMEM((PBLK,), jnp.int32),
            pltpu.VMEM((EPT + CHUNK,), jnp.int32),
            pltpu.VMEM((EPT + CHUNK,), jnp.float32),
            pltpu.VMEM((LANES,), jnp.int32),
        ],
        **_SC_PARAMS,
    )(dinv_flat, gs_x, gd_x, es_x, row_flat)


def _msg_body(tab_hbm, pidx_hbm, w_hbm, cnt_hbm, s_hbm,
              pb, gb2, rb2, sb2, cb16, rows, acc, sem, ssem):
    c = lax.axis_index("c")
    s = lax.axis_index("s")
    zv = jnp.zeros((LANES,), jnp.float32)

    def _zrow(j, carry):
        for q in range(D // LANES):
            rows[0, j, pl.ds(q * LANES, LANES)] = zv
        return carry

    lax.fori_loop(0, CHUNK, _zrow, 0)

    def _zstripe(i, carry):
        pltpu.sync_copy(rows.at[0], acc.at[pl.ds(s * RPT + i * CHUNK, CHUNK)])
        return carry

    lax.fori_loop(0, RPT // CHUNK, _zstripe, 0)
    if RPT % CHUNK:
        pltpu.sync_copy(
            rows.at[0, pl.ds(0, RPT % CHUNK)],
            acc.at[pl.ds(s * RPT + (RPT // CHUNK) * CHUNK, RPT % CHUNK)])
    plsc.subcore_barrier()

    pltpu.sync_copy(cnt_hbm.at[pl.ds((c * NS + s) * LANES, LANES)], cb16)
    n = cb16[pl.ds(0, LANES)][0]
    nch = (n + (CHUNK - 1)) // CHUNK
    base = c * ET + s * EPT

    def _issue(ci, b):
        @pl.when(ci >= 2)
        def _():
            pltpu.make_async_copy(rows.at[b], acc.at[rb2.at[b]],
                                  ssem.at[b]).wait()
        off = base + ci * CHUNK
        pltpu.sync_copy(pidx_hbm.at[pl.ds(off, CHUNK)], pb)
        pltpu.sync_copy(w_hbm.at[pl.ds(off, CHUNK)],
                        sb2.at[b])

        def _unpack(j, carry):
            q = j * LANES
            pv = pb[pl.ds(q, LANES)]
            gb2[b, pl.ds(q, LANES)] = jnp.right_shift(pv, 14)
            rb2[b, pl.ds(q, LANES)] = jnp.bitwise_and(pv, 16383)
            return carry

        lax.fori_loop(0, CHUNK // LANES, _unpack, 0)
        pltpu.async_copy(tab_hbm.at[gb2.at[b]], rows.at[b], sem.at[b])

    @pl.when(nch > 0)
    def _():
        _issue(jnp.int32(0), jnp.int32(0))

    def _chunk(ci, carry):
        b = lax.rem(ci, 2)

        @pl.when(ci + 1 < nch)
        def _():
            _issue(ci + 1, 1 - b)

        pltpu.make_async_copy(tab_hbm.at[gb2.at[b]], rows.at[b],
                              sem.at[b]).wait()

        def _scale(j, carry2):
            w = plsc.load_gather(
                sb2, [jnp.full((LANES,), b, jnp.int32),
                      jnp.full((LANES,), j, jnp.int32)])
            for q in range(D // LANES):
                rows[b, j, pl.ds(q * LANES, LANES)] = (
                    rows[b, j, pl.ds(q * LANES, LANES)] * w)
            return carry2

        lax.fori_loop(0, CHUNK, _scale, 0)
        pltpu.async_copy(rows.at[b], acc.at[rb2.at[b]], ssem.at[b], add=True)
        return carry

    lax.fori_loop(0, nch, _chunk, 0)

    def _drain(k, carry):
        @pl.when(nch > k)
        def _():
            b = lax.rem(nch - 1 - k, 2)
            pltpu.make_async_copy(rows.at[b], acc.at[rb2.at[b]],
                                  ssem.at[b]).wait()
        return carry

    lax.fori_loop(0, 2, _drain, 0)
    plsc.subcore_barrier()
    pltpu.sync_copy(acc.at[pl.ds(s * RPT, RPT)], s_hbm.at[c, pl.ds(s * RPT, RPT)])


def _msg_call(tab_flat, pidx, w_c, cnt):
    return pl.kernel(
        _msg_body,
        out_type=jax.ShapeDtypeStruct((NC, SPAD, D), jnp.float32),
        mesh=_sc_mesh(),
        scratch_types=[
            pltpu.VMEM((CHUNK,), jnp.int32),
            pltpu.VMEM((2, CHUNK), jnp.int32),
            pltpu.VMEM((2, CHUNK), jnp.int32),
            pltpu.VMEM((2, CHUNK), jnp.float32),
            pltpu.VMEM((LANES,), jnp.int32),
            pltpu.VMEM((2, CHUNK, D), jnp.float32),
            pltpu.VMEM_SHARED((SPAD, D), jnp.float32),
            pltpu.SemaphoreType.DMA((2,)),
            pltpu.SemaphoreType.DMA((2,)),
        ],
        **_SC_PARAMS,
    )(tab_flat, pidx, w_c, cnt)


def _tc1_body(x_ref, pw_ref, pb_ref, bg_ref, bb_ref, parts_ref,
              src_ref, dst_ref, et_ref, cw_ref,
              h_ref, dinv_ref, gs_ref, gd_ref, row_ref, tab_ref):
    z = jnp.dot(x_ref[...], pw_ref[...],
                preferred_element_type=jnp.float32) + pb_ref[0][None, :]
    mu = jnp.mean(z, axis=0, keepdims=True)
    zc = z - mu
    var = jnp.mean(zc * zc, axis=0, keepdims=True)
    h = bg_ref[0][None, :] * zc / jnp.sqrt(var + EPS) + bb_ref[0][None, :]
    h = jnp.maximum(h, 0.0)
    h_ref[...] = h
    for r in range(NREL):
        tab_ref[r] = jnp.dot(h, cw_ref[r], preferred_element_type=jnp.float32)
    deg = jnp.sum(parts_ref[...], axis=0, keepdims=True)
    dinv_ref[...] = lax.rsqrt(deg)
    sv = src_ref[...]
    dv = dst_ref[...]
    tn = et_ref[...] * N
    gs_ref[...] = tn + sv
    gd_ref[...] = tn + dv
    th = et_ref[...] * HALF
    row_ref[0] = jnp.where(dv < HALF, th + dv, DUMMY)
    row_ref[1] = jnp.where(dv >= HALF, th + (dv - HALF), DUMMY)


def _comb_body(h_ref, s3_ref, skw_ref, skb_ref, cb_ref, o_ref):
    out = jnp.dot(h_ref[...], skw_ref[...],
                  preferred_element_type=jnp.float32) + skb_ref[0][None, :]
    for r in range(NREL):
        out = out + s3_ref[r] + cb_ref[r][None, :]
    o_ref[...] = out


def _bn_relu_body(z_ref, bg_ref, bb_ref, o_ref):
    z = z_ref[...]
    mu = jnp.mean(z, axis=0, keepdims=True)
    zc = z - mu
    var = jnp.mean(zc * zc, axis=0, keepdims=True)
    o = bg_ref[0][None, :] * zc / jnp.sqrt(var + EPS) + bb_ref[0][None, :]
    o_ref[...] = jnp.maximum(o, 0.0)


def _bn_relu_tab_body(z_ref, bg_ref, bb_ref, cw_ref, o_ref, tab_ref):
    z = z_ref[...]
    mu = jnp.mean(z, axis=0, keepdims=True)
    zc = z - mu
    var = jnp.mean(zc * zc, axis=0, keepdims=True)
    o = bg_ref[0][None, :] * zc / jnp.sqrt(var + EPS) + bb_ref[0][None, :]
    o = jnp.maximum(o, 0.0)
    o_ref[...] = o
    for r in range(NREL):
        tab_ref[r] = jnp.dot(o, cw_ref[r], preferred_element_type=jnp.float32)


def _layer_call(h, s_acc, skw, skb, cb, bng, bnb, cw_next):
    s3 = jnp.stack(
        [jnp.concatenate([s_acc[0, r * HALF:(r + 1) * HALF, :],
                          s_acc[1, r * HALF:(r + 1) * HALF, :]], axis=0)
         for r in range(NREL)])
    z = pl.pallas_call(
        _comb_body,
        out_shape=jax.ShapeDtypeStruct((N, D), jnp.float32),
    )(h, s3, skw, skb, cb)
    if cw_next is None:
        h2 = pl.pallas_call(
            _bn_relu_body,
            out_shape=jax.ShapeDtypeStruct((N, D), jnp.float32),
        )(z, bng, bnb)
        return h2, None
    h2, tab = pl.pallas_call(
        _bn_relu_tab_body,
        out_shape=[jax.ShapeDtypeStruct((N, D), jnp.float32),
                   jax.ShapeDtypeStruct((NREL, N, D), jnp.float32)],
    )(z, bng, bnb, cw_next)
    return h2, tab


def _tc3_body(h_ref, b_ref, w1_ref, b1_ref, g1_ref, c1_ref,
              w2_ref, b2_ref, g2_ref, c2_ref, w3_ref, b3_ref, o_ref):
    gi = lax.broadcasted_iota(jnp.int32, (N, NG), 1)
    onehot = (b_ref[...] == gi).astype(jnp.float32)
    pooled = lax.dot_general(onehot, h_ref[...], (((0,), (0,)), ((), ())),
                             preferred_element_type=jnp.float32, precision=lax.Precision.HIGHEST)

    def bn(t, g_ref, c_ref):
        mu = jnp.mean(t, axis=0, keepdims=True)
        tc = t - mu
        var = jnp.mean(tc * tc, axis=0, keepdims=True)
        return g_ref[0][None, :] * tc / jnp.sqrt(var + EPS) + c_ref[0][None, :]

    z1 = jnp.dot(pooled, w1_ref[...],
                 preferred_element_type=jnp.float32) + b1_ref[0][None, :]
    z1 = jnp.maximum(bn(z1, g1_ref, c1_ref), 0.0)
    z2 = jnp.dot(z1, w2_ref[...],
                 preferred_element_type=jnp.float32) + b2_ref[0][None, :]
    z2 = jnp.maximum(bn(z2, g2_ref, c2_ref), 0.0)
    o_ref[...] = jnp.dot(z2, w3_ref[...],
                         preferred_element_type=jnp.float32) + b3_ref[0][None, :]


def kernel(x, edge_index, edge_type, edge_score, batch,
           pre_W, pre_b, pre_bn_g, pre_bn_b,
           conv_W, conv_b, skip_W, skip_b, layer_bn_g, layer_bn_b,
           post_W1, post_b1, post_bn1_g, post_bn1_b,
           post_W2, post_b2, post_bn2_g, post_bn2_b,
           post_W3, post_b3):
    i32 = jnp.int32
    f32 = jnp.float32
    src = edge_index[0].astype(i32)
    dst = edge_index[1].astype(i32)
    et = edge_type.astype(i32)
    es = edge_score.astype(f32)

    loop = jnp.arange(N, dtype=i32)
    padi = jnp.zeros((PADE,), i32)
    src_x = jnp.concatenate([src, loop, loop, loop, padi])
    dst_x = jnp.concatenate([dst, loop, loop, loop, padi])
    et_x = jnp.concatenate([et, jnp.zeros((N,), i32), jnp.ones((N,), i32),
                            jnp.full((N,), 2, i32), padi])
    es_x = jnp.concatenate([es, jnp.ones((NREL * N,), f32),
                            jnp.zeros((PADE,), f32)])

    parts = _deg_call(dst_x, et_x, es_x)

    h, dinv, gs, gd, rowcat, tab = pl.pallas_call(
        _tc1_body,
        out_shape=[jax.ShapeDtypeStruct((N, D), f32),
                   jax.ShapeDtypeStruct((1, NREL * N), f32),
                   jax.ShapeDtypeStruct((ETR, 128), i32),
                   jax.ShapeDtypeStruct((ETR, 128), i32),
                   jax.ShapeDtypeStruct((2, ETR, 128), i32),
                   jax.ShapeDtypeStruct((NREL, N, D), f32)],
    )(x, pre_W, pre_b.reshape(1, D), pre_bn_g.reshape(1, D),
      pre_bn_b.reshape(1, D), parts,
      src_x.reshape(ETR, 128), dst_x.reshape(ETR, 128), et_x.reshape(ETR, 128),
      conv_W[0])

    gs_flat = gs.reshape(ET)
    row_flat = rowcat.reshape(2 * ET)
    pidx, w_c, cnt = _prep_call(dinv.reshape(NREL * N), gs_flat,
                                gd.reshape(ET), es_x, row_flat)

    for l in range(2):
        s_acc = _msg_call(tab.reshape(NREL * N, D), pidx, w_c, cnt)
        cw_next = conv_W[1] if l == 0 else None
        h, tab = _layer_call(h, s_acc, skip_W[l], skip_b[l].reshape(1, D),
                             conv_b[l],
                             layer_bn_g[l].reshape(1, D),
                             layer_bn_b[l].reshape(1, D), cw_next)

    out = pl.pallas_call(
        _tc3_body,
        out_shape=jax.ShapeDtypeStruct((NG, 16), f32),
    )(h, batch.astype(i32).reshape(N, 1),
      post_W1, post_b1.reshape(1, D), post_bn1_g.reshape(1, D),
      post_bn1_b.reshape(1, D),
      post_W2, post_b2.reshape(1, 32), post_bn2_g.reshape(1, 32),
      post_bn2_b.reshape(1, 32),
      post_W3, post_b3.reshape(1, 16))
    return out


# speedup vs baseline: 21.3989x; 1.1898x over previous
import jax
import jax.numpy as jnp
from jax import lax
from jax.experimental import pallas as pl
from jax.experimental.pallas import tpu as pltpu
from jax.experimental.pallas import tpu_sc as plsc

N = 10000
E = 320000
NREL = 3
D = 128
NG = 64
EPS = 1e-5

NC = 2
NS = 16
LANES = 16
HALF = N // NC
DUMMY = NREL * HALF
SPAD = 15104
RPT = SPAD // NS
CHUNK = 32
ET = 353280
PADE = ET - E - NREL * N
ETR = ET // 128
EPT = ET // NS
EPW = ET // (NC * NS)

_SC_PARAMS = dict(
    compiler_params=pltpu.CompilerParams(needs_layout_passes=False),
)


def _sc_mesh():
    return plsc.VectorSubcoreMesh(core_axis_name="c", subcore_axis_name="s")


def _deg_body(dst_hbm, et_hbm, es_hbm, part_hbm, dstb, etb, esb, accb):
    c = lax.axis_index("c")
    s = lax.axis_index("s")
    wid = c * NS + s
    base = wid * EPW
    pltpu.sync_copy(dst_hbm.at[pl.ds(base, EPW)], dstb)
    pltpu.sync_copy(et_hbm.at[pl.ds(base, EPW)], etb)
    pltpu.sync_copy(es_hbm.at[pl.ds(base, EPW)], esb)
    zv = jnp.zeros((LANES,), jnp.float32)

    def _zero(i, carry):
        accb[pl.ds(i * LANES, LANES)] = zv
        return carry

    lax.fori_loop(0, NREL * N // LANES, _zero, 0)

    def _edge(i, carry):
        t = etb[pl.ds(i * LANES, LANES)]
        d = dstb[pl.ds(i * LANES, LANES)]
        w = esb[pl.ds(i * LANES, LANES)]
        plsc.addupdate_scatter(accb, [t * N + d], w)
        return carry

    lax.fori_loop(0, EPW // LANES, _edge, 0)
    pltpu.sync_copy(accb, part_hbm.at[wid])


def _deg_call(dst_x, et_x, es_x):
    return pl.kernel(
        _deg_body,
        out_type=jax.ShapeDtypeStruct((NC * NS, NREL * N), jnp.float32),
        mesh=_sc_mesh(),
        scratch_types=[
            pltpu.VMEM((EPW,), jnp.int32),
            pltpu.VMEM((EPW,), jnp.int32),
            pltpu.VMEM((EPW,), jnp.float32),
            pltpu.VMEM((NREL * N,), jnp.float32),
        ],
        **_SC_PARAMS,
    )(dst_x, et_x, es_x)


PBLK = 5520


def _prep_body(dinv_hbm, gs_hbm, gd_hbm, es_hbm, row_hbm, pidx_hbm, w_hbm,
               cnt_hbm, dinvb, gsb, gdb, esb, rwb, pout, wout, cntb):
    c = lax.axis_index("c")
    s = lax.axis_index("s")
    wbase = s * EPT
    pltpu.sync_copy(dinv_hbm, dinvb)

    def _block(bi, o):
        bbase = wbase + bi * PBLK
        pltpu.sync_copy(gs_hbm.at[pl.ds(bbase, PBLK)], gsb)
        pltpu.sync_copy(gd_hbm.at[pl.ds(bbase, PBLK)], gdb)
        pltpu.sync_copy(es_hbm.at[pl.ds(bbase, PBLK)], esb)
        pltpu.sync_copy(row_hbm.at[pl.ds(c * ET + bbase, PBLK)], rwb)

        def _vec(i, o2):
            q = i * LANES
            gsv = gsb[pl.ds(q, LANES)]
            gdv = gdb[pl.ds(q, LANES)]
            esv = esb[pl.ds(q, LANES)]
            rwv = rwb[pl.ds(q, LANES)]
            w = (esv * plsc.load_gather(dinvb, [gsv])
                 * plsc.load_gather(dinvb, [gdv]))
            packed = jnp.bitwise_or(jnp.left_shift(gsv, 14), rwv)
            mask = rwv != DUMMY
            plsc.store_compressed(pout.at[pl.ds(o2, LANES)], packed, mask=mask)
            plsc.store_compressed(wout.at[pl.ds(o2, LANES)], w, mask=mask)
            cntv = plsc.all_reduce_population_count(mask)
            return o2 + cntv[0]

        return lax.fori_loop(0, PBLK // LANES, _vec, o)

    n = lax.fori_loop(0, EPT // PBLK, _block, jnp.int32(0))

    padv = jnp.full((LANES,), DUMMY, jnp.int32)
    zv = jnp.zeros((LANES,), jnp.float32)
    npad = (CHUNK - 1) - ((n + CHUNK - 1) % CHUNK)

    def _pad(i, carry):
        @pl.when(i * LANES < npad)
        def _():
            pout[pl.ds(n + i * LANES, LANES)] = padv
            wout[pl.ds(n + i * LANES, LANES)] = zv
        return carry

    lax.fori_loop(0, CHUNK // LANES, _pad, 0)

    cntb[pl.ds(0, LANES)] = jnp.full((LANES,), n, jnp.int32)
    base = (c * NS + s)
    pltpu.sync_copy(pout.at[pl.ds(0, EPT)], pidx_hbm.at[pl.ds(c * ET + wbase, EPT)])
    pltpu.sync_copy(wout.at[pl.ds(0, EPT)], w_hbm.at[pl.ds(c * ET + wbase, EPT)])
    pltpu.sync_copy(cntb, cnt_hbm.at[pl.ds(base * LANES, LANES)])


def _prep_call(dinv_flat, gs_x, gd_x, es_x, row_flat):
    return pl.kernel(
        _prep_body,
        out_type=[jax.ShapeDtypeStruct((NC * ET,), jnp.int32),
                  jax.ShapeDtypeStruct((NC * ET,), jnp.float32),
                  jax.ShapeDtypeStruct((NC * NS * LANES,), jnp.int32)],
        mesh=_sc_mesh(),
        scratch_types=[
            pltpu.VMEM((NREL * N,), jnp.float32),
            pltpu.VMEM((PBLK,), jnp.int32),
            pltpu.VMEM((PBLK,), jnp.int32),
            pltpu.VMEM((PBLK,), jnp.float32),
            pltpu.VMEM((PBLK,), jnp.int32),
            pltpu.VMEM((EPT + CHUNK,), jnp.int32),
            pltpu.VMEM((EPT + CHUNK,), jnp.float32),
            pltpu.VMEM((LANES,), jnp.int32),
        ],
        **_SC_PARAMS,
    )(dinv_flat, gs_x, gd_x, es_x, row_flat)


def _msg_body(tab_hbm, pidx_hbm, w_hbm, cnt_hbm, s_hbm,
              pb, gb2, rb2, sb2, cb16, rows, acc, sem, ssem, csem):
    c = lax.axis_index("c")
    s = lax.axis_index("s")
    zv = jnp.zeros((LANES,), jnp.float32)

    def _zrow(j, carry):
        for q in range(D // LANES):
            rows[0, j, pl.ds(q * LANES, LANES)] = zv
        return carry

    lax.fori_loop(0, CHUNK, _zrow, 0)

    def _zstripe(i, carry):
        pltpu.sync_copy(rows.at[0], acc.at[pl.ds(s * RPT + i * CHUNK, CHUNK)])
        return carry

    lax.fori_loop(0, RPT // CHUNK, _zstripe, 0)
    if RPT % CHUNK:
        pltpu.sync_copy(
            rows.at[0, pl.ds(0, RPT % CHUNK)],
            acc.at[pl.ds(s * RPT + (RPT // CHUNK) * CHUNK, RPT % CHUNK)])
    plsc.subcore_barrier()

    pltpu.sync_copy(cnt_hbm.at[pl.ds((c * NS + s) * LANES, LANES)], cb16)
    n = cb16[pl.ds(0, LANES)][0]
    nch = (n + (CHUNK - 1)) // CHUNK
    base = c * ET + s * EPT

    def _issue(ci, b):
        @pl.when(ci >= 2)
        def _():
            pltpu.make_async_copy(rows.at[b], acc.at[rb2.at[b]],
                                  ssem.at[b]).wait()
        off = base + ci * CHUNK
        pltpu.async_copy(pidx_hbm.at[pl.ds(off, CHUNK)], pb, csem)
        pltpu.async_copy(w_hbm.at[pl.ds(off, CHUNK)], sb2.at[b], csem)
        pltpu.make_async_copy(pidx_hbm.at[pl.ds(off, CHUNK)], pb, csem).wait()
        pltpu.make_async_copy(w_hbm.at[pl.ds(off, CHUNK)], sb2.at[b],
                              csem).wait()

        def _unpack(j, carry):
            q = j * LANES
            pv = pb[pl.ds(q, LANES)]
            gb2[b, pl.ds(q, LANES)] = jnp.right_shift(pv, 14)
            rb2[b, pl.ds(q, LANES)] = jnp.bitwise_and(pv, 16383)
            return carry

        lax.fori_loop(0, CHUNK // LANES, _unpack, 0)
        pltpu.async_copy(tab_hbm.at[gb2.at[b]], rows.at[b], sem.at[b])

    @pl.when(nch > 0)
    def _():
        _issue(jnp.int32(0), jnp.int32(0))

    def _chunk(ci, carry):
        b = lax.rem(ci, 2)

        @pl.when(ci + 1 < nch)
        def _():
            _issue(ci + 1, 1 - b)

        pltpu.make_async_copy(tab_hbm.at[gb2.at[b]], rows.at[b],
                              sem.at[b]).wait()

        def _scale(j, carry2):
            w = plsc.load_gather(
                sb2, [jnp.full((LANES,), b, jnp.int32),
                      jnp.full((LANES,), j, jnp.int32)])
            for q in range(D // LANES):
                rows[b, j, pl.ds(q * LANES, LANES)] = (
                    rows[b, j, pl.ds(q * LANES, LANES)] * w)
            return carry2

        lax.fori_loop(0, CHUNK, _scale, 0)
        pltpu.async_copy(rows.at[b], acc.at[rb2.at[b]], ssem.at[b], add=True)
        return carry

    lax.fori_loop(0, nch, _chunk, 0)

    def _drain(k, carry):
        @pl.when(nch > k)
        def _():
            b = lax.rem(nch - 1 - k, 2)
            pltpu.make_async_copy(rows.at[b], acc.at[rb2.at[b]],
                                  ssem.at[b]).wait()
        return carry

    lax.fori_loop(0, 2, _drain, 0)
    plsc.subcore_barrier()
    pltpu.sync_copy(acc.at[pl.ds(s * RPT, RPT)], s_hbm.at[c, pl.ds(s * RPT, RPT)])


def _msg_call(tab_flat, pidx, w_c, cnt):
    return pl.kernel(
        _msg_body,
        out_type=jax.ShapeDtypeStruct((NC, SPAD, D), jnp.float32),
        mesh=_sc_mesh(),
        scratch_types=[
            pltpu.VMEM((CHUNK,), jnp.int32),
            pltpu.VMEM((2, CHUNK), jnp.int32),
            pltpu.VMEM((2, CHUNK), jnp.int32),
            pltpu.VMEM((2, CHUNK), jnp.float32),
            pltpu.VMEM((LANES,), jnp.int32),
            pltpu.VMEM((2, CHUNK, D), jnp.float32),
            pltpu.VMEM_SHARED((SPAD, D), jnp.float32),
            pltpu.SemaphoreType.DMA((2,)),
            pltpu.SemaphoreType.DMA((2,)),
            pltpu.SemaphoreType.DMA,
        ],
        **_SC_PARAMS,
    )(tab_flat, pidx, w_c, cnt)


def _tc1_body(x_ref, pw_ref, pb_ref, bg_ref, bb_ref, parts_ref,
              src_ref, dst_ref, et_ref, cw_ref,
              h_ref, dinv_ref, gs_ref, gd_ref, row_ref, tab_ref):
    z = jnp.dot(x_ref[...], pw_ref[...],
                preferred_element_type=jnp.float32) + pb_ref[0][None, :]
    mu = jnp.mean(z, axis=0, keepdims=True)
    zc = z - mu
    var = jnp.mean(zc * zc, axis=0, keepdims=True)
    h = bg_ref[0][None, :] * zc / jnp.sqrt(var + EPS) + bb_ref[0][None, :]
    h = jnp.maximum(h, 0.0)
    h_ref[...] = h
    for r in range(NREL):
        tab_ref[r] = jnp.dot(h, cw_ref[r], preferred_element_type=jnp.float32)
    deg = jnp.sum(parts_ref[...], axis=0, keepdims=True)
    dinv_ref[...] = lax.rsqrt(deg)
    sv = src_ref[...]
    dv = dst_ref[...]
    tn = et_ref[...] * N
    gs_ref[...] = tn + sv
    gd_ref[...] = tn + dv
    th = et_ref[...] * HALF
    row_ref[0] = jnp.where(dv < HALF, th + dv, DUMMY)
    row_ref[1] = jnp.where(dv >= HALF, th + (dv - HALF), DUMMY)


def _comb_body(h_ref, s3_ref, skw_ref, skb_ref, cb_ref, o_ref):
    out = jnp.dot(h_ref[...], skw_ref[...],
                  preferred_element_type=jnp.float32) + skb_ref[0][None, :]
    for r in range(NREL):
        out = out + s3_ref[r] + cb_ref[r][None, :]
    o_ref[...] = out


def _bn_relu_body(z_ref, bg_ref, bb_ref, o_ref):
    z = z_ref[...]
    mu = jnp.mean(z, axis=0, keepdims=True)
    zc = z - mu
    var = jnp.mean(zc * zc, axis=0, keepdims=True)
    o = bg_ref[0][None, :] * zc / jnp.sqrt(var + EPS) + bb_ref[0][None, :]
    o_ref[...] = jnp.maximum(o, 0.0)


def _bn_relu_tab_body(z_ref, bg_ref, bb_ref, cw_ref, o_ref, tab_ref):
    z = z_ref[...]
    mu = jnp.mean(z, axis=0, keepdims=True)
    zc = z - mu
    var = jnp.mean(zc * zc, axis=0, keepdims=True)
    o = bg_ref[0][None, :] * zc / jnp.sqrt(var + EPS) + bb_ref[0][None, :]
    o = jnp.maximum(o, 0.0)
    o_ref[...] = o
    for r in range(NREL):
        tab_ref[r] = jnp.dot(o, cw_ref[r], preferred_element_type=jnp.float32)


def _layer_call(h, s_acc, skw, skb, cb, bng, bnb, cw_next):
    s3 = jnp.stack(
        [jnp.concatenate([s_acc[0, r * HALF:(r + 1) * HALF, :],
                          s_acc[1, r * HALF:(r + 1) * HALF, :]], axis=0)
         for r in range(NREL)])
    z = pl.pallas_call(
        _comb_body,
        out_shape=jax.ShapeDtypeStruct((N, D), jnp.float32),
    )(h, s3, skw, skb, cb)
    if cw_next is None:
        h2 = pl.pallas_call(
            _bn_relu_body,
            out_shape=jax.ShapeDtypeStruct((N, D), jnp.float32),
        )(z, bng, bnb)
        return h2, None
    h2, tab = pl.pallas_call(
        _bn_relu_tab_body,
        out_shape=[jax.ShapeDtypeStruct((N, D), jnp.float32),
                   jax.ShapeDtypeStruct((NREL, N, D), jnp.float32)],
    )(z, bng, bnb, cw_next)
    return h2, tab


def _tc3_body(h_ref, b_ref, w1_ref, b1_ref, g1_ref, c1_ref,
              w2_ref, b2_ref, g2_ref, c2_ref, w3_ref, b3_ref, o_ref):
    gi = lax.broadcasted_iota(jnp.int32, (N, NG), 1)
    onehot = (b_ref[...] == gi).astype(jnp.float32)
    pooled = lax.dot_general(onehot, h_ref[...], (((0,), (0,)), ((), ())),
                             preferred_element_type=jnp.float32, precision=lax.Precision.HIGHEST)

    def bn(t, g_ref, c_ref):
        mu = jnp.mean(t, axis=0, keepdims=True)
        tc = t - mu
        var = jnp.mean(tc * tc, axis=0, keepdims=True)
        return g_ref[0][None, :] * tc / jnp.sqrt(var + EPS) + c_ref[0][None, :]

    z1 = jnp.dot(pooled, w1_ref[...],
                 preferred_element_type=jnp.float32) + b1_ref[0][None, :]
    z1 = jnp.maximum(bn(z1, g1_ref, c1_ref), 0.0)
    z2 = jnp.dot(z1, w2_ref[...],
                 preferred_element_type=jnp.float32) + b2_ref[0][None, :]
    z2 = jnp.maximum(bn(z2, g2_ref, c2_ref), 0.0)
    o_ref[...] = jnp.dot(z2, w3_ref[...],
                         preferred_element_type=jnp.float32) + b3_ref[0][None, :]


def kernel(x, edge_index, edge_type, edge_score, batch,
           pre_W, pre_b, pre_bn_g, pre_bn_b,
           conv_W, conv_b, skip_W, skip_b, layer_bn_g, layer_bn_b,
           post_W1, post_b1, post_bn1_g, post_bn1_b,
           post_W2, post_b2, post_bn2_g, post_bn2_b,
           post_W3, post_b3):
    i32 = jnp.int32
    f32 = jnp.float32
    src = edge_index[0].astype(i32)
    dst = edge_index[1].astype(i32)
    et = edge_type.astype(i32)
    es = edge_score.astype(f32)

    loop = jnp.arange(N, dtype=i32)
    padi = jnp.zeros((PADE,), i32)
    src_x = jnp.concatenate([src, loop, loop, loop, padi])
    dst_x = jnp.concatenate([dst, loop, loop, loop, padi])
    et_x = jnp.concatenate([et, jnp.zeros((N,), i32), jnp.ones((N,), i32),
                            jnp.full((N,), 2, i32), padi])
    es_x = jnp.concatenate([es, jnp.ones((NREL * N,), f32),
                            jnp.zeros((PADE,), f32)])

    parts = _deg_call(dst_x, et_x, es_x)

    h, dinv, gs, gd, rowcat, tab = pl.pallas_call(
        _tc1_body,
        out_shape=[jax.ShapeDtypeStruct((N, D), f32),
                   jax.ShapeDtypeStruct((1, NREL * N), f32),
                   jax.ShapeDtypeStruct((ETR, 128), i32),
                   jax.ShapeDtypeStruct((ETR, 128), i32),
                   jax.ShapeDtypeStruct((2, ETR, 128), i32),
                   jax.ShapeDtypeStruct((NREL, N, D), f32)],
    )(x, pre_W, pre_b.reshape(1, D), pre_bn_g.reshape(1, D),
      pre_bn_b.reshape(1, D), parts,
      src_x.reshape(ETR, 128), dst_x.reshape(ETR, 128), et_x.reshape(ETR, 128),
      conv_W[0])

    gs_flat = gs.reshape(ET)
    row_flat = rowcat.reshape(2 * ET)
    pidx, w_c, cnt = _prep_call(dinv.reshape(NREL * N), gs_flat,
                                gd.reshape(ET), es_x, row_flat)

    for l in range(2):
        s_acc = _msg_call(tab.reshape(NREL * N, D), pidx, w_c, cnt)
        cw_next = conv_W[1] if l == 0 else None
        h, tab = _layer_call(h, s_acc, skip_W[l], skip_b[l].reshape(1, D),
                             conv_b[l],
                             layer_bn_g[l].reshape(1, D),
                             layer_bn_b[l].reshape(1, D), cw_next)

    out = pl.pallas_call(
        _tc3_body,
        out_shape=jax.ShapeDtypeStruct((NG, 16), f32),
    )(h, batch.astype(i32).reshape(N, 1),
      post_W1, post_b1.reshape(1, D), post_bn1_g.reshape(1, D),
      post_bn1_b.reshape(1, D),
      post_W2, post_b2.reshape(1, 32), post_bn2_g.reshape(1, 32),
      post_bn2_b.reshape(1, 32),
      post_W3, post_b3.reshape(1, 16))
    return out
